# Initial kernel scaffold; baseline (speedup 1.0000x reference)
import jax
import jax.numpy as jnp
from jax.experimental import pallas as pl


def kernel(self_input, self_adj, view2_input, view2_adj, view3_input, view3_adj, weight_self, weight_view2, weight_view3, weight_all_views, bias):
    raise NotImplementedError("write your pallas kernel here")


# speedup vs baseline: 1.1330x; 1.1330x over previous
import jax
import jax.numpy as jnp
from jax.experimental import pallas as pl
from jax.experimental.pallas import tpu as pltpu

_N = 10000
_F = 128
_BM = 128


def _embed_body(x1_ref, x2_ref, x3_ref, w1_ref, w2_ref, w3_ref, scale_ref,
                bias_ref, base_ref, s1_ref, s2_ref, s3_ref):
    e1 = jnp.dot(x1_ref[...], w1_ref[...], preferred_element_type=jnp.float32,
                 precision=jax.lax.Precision.HIGHEST)
    base_ref[...] = e1 + bias_ref[...]
    s1_ref[...] = e1 * scale_ref[...][0:1, 0:1]
    e2 = jnp.dot(x2_ref[...], w2_ref[...], preferred_element_type=jnp.float32,
                 precision=jax.lax.Precision.HIGHEST)
    s2_ref[...] = e2 * scale_ref[...][0:1, 1:2]
    e3 = jnp.dot(x3_ref[...], w3_ref[...], preferred_element_type=jnp.float32,
                 precision=jax.lax.Precision.HIGHEST)
    s3_ref[...] = e3 * scale_ref[...][0:1, 2:3]


def _agg_body(a1_ref, a2_ref, a3_ref, s1_ref, s2_ref, s3_ref, base_ref,
              out_ref):
    acc = jnp.dot(a1_ref[...], s1_ref[...], preferred_element_type=jnp.float32,
                  precision=jax.lax.Precision.DEFAULT)
    acc = acc + jnp.dot(a2_ref[...], s2_ref[...],
                        preferred_element_type=jnp.float32,
                        precision=jax.lax.Precision.DEFAULT)
    acc = acc + jnp.dot(a3_ref[...], s3_ref[...],
                        preferred_element_type=jnp.float32,
                        precision=jax.lax.Precision.DEFAULT)
    out_ref[...] = acc + base_ref[...]


def kernel(self_input, self_adj, view2_input, view2_adj, view3_input,
           view3_adj, weight_self, weight_view2, weight_view3,
           weight_all_views, bias):
    scale = (1.01 * weight_all_views).astype(jnp.float32)
    bias2d = bias.reshape(1, _F).astype(jnp.float32)

    bm_e = 2000
    row_e = pl.BlockSpec((bm_e, _F), lambda i: (i, 0))
    wspec = pl.BlockSpec((_F, _F), lambda *_: (0, 0))

    base, s1, s2, s3 = pl.pallas_call(
        _embed_body,
        grid=(_N // bm_e,),
        in_specs=[row_e, row_e, row_e, wspec, wspec, wspec,
                  pl.BlockSpec((1, 3), lambda *_: (0, 0)),
                  pl.BlockSpec((1, _F), lambda *_: (0, 0))],
        out_specs=[row_e, row_e, row_e, row_e],
        out_shape=[jax.ShapeDtypeStruct((_N, _F), jnp.float32)] * 4,
    )(self_input, view2_input, view3_input, weight_self, weight_view2,
      weight_view3, scale, bias2d)

    nblocks = pl.cdiv(_N, _BM)
    adj_spec = pl.BlockSpec((_BM, _N), lambda i: (i, 0))
    emb_spec = pl.BlockSpec((_N, _F), lambda i: (0, 0))
    row_spec = pl.BlockSpec((_BM, _F), lambda i: (i, 0))

    out = pl.pallas_call(
        _agg_body,
        grid=(nblocks,),
        in_specs=[adj_spec, adj_spec, adj_spec, emb_spec, emb_spec, emb_spec,
                  row_spec],
        out_specs=row_spec,
        out_shape=jax.ShapeDtypeStruct((_N, _F), jnp.float32),
        compiler_params=pltpu.CompilerParams(
            dimension_semantics=("arbitrary",),
        ),
    )(self_adj, view2_adj, view3_adj, s1, s2, s3, base)

    return out


# speedup vs baseline: 1.1439x; 1.0096x over previous
import jax
import jax.numpy as jnp
from jax.experimental import pallas as pl
from jax.experimental.pallas import tpu as pltpu

_N = 10000
_F = 128
_BM = 80


def _fused_body(x1_ref, x2_ref, x3_ref, w1_ref, w1s_ref, w2s_ref, w3s_ref,
                bias_ref, a1_ref, a2_ref, a3_ref, out_ref,
                s1_ref, s2_ref, s3_ref):
    i = pl.program_id(0)

    @pl.when(i == 0)
    def _():
        cb = 2000

        def chunk(j, carry):
            sl = pl.ds(j * cb, cb)
            s1_ref[sl, :] = jnp.dot(x1_ref[sl, :], w1s_ref[...],
                                    preferred_element_type=jnp.float32,
                                    precision=jax.lax.Precision.HIGHEST)
            s2_ref[sl, :] = jnp.dot(x2_ref[sl, :], w2s_ref[...],
                                    preferred_element_type=jnp.float32,
                                    precision=jax.lax.Precision.HIGHEST)
            s3_ref[sl, :] = jnp.dot(x3_ref[sl, :], w3s_ref[...],
                                    preferred_element_type=jnp.float32,
                                    precision=jax.lax.Precision.HIGHEST)
            return carry

        jax.lax.fori_loop(0, _N // cb, chunk, 0)

    acc = jnp.dot(a1_ref[...], s1_ref[...], preferred_element_type=jnp.float32,
                  precision=jax.lax.Precision.DEFAULT)
    acc = acc + jnp.dot(a2_ref[...], s2_ref[...],
                        preferred_element_type=jnp.float32,
                        precision=jax.lax.Precision.DEFAULT)
    acc = acc + jnp.dot(a3_ref[...], s3_ref[...],
                        preferred_element_type=jnp.float32,
                        precision=jax.lax.Precision.DEFAULT)
    base = jnp.dot(x1_ref[pl.ds(i * _BM, _BM), :], w1_ref[...],
                   preferred_element_type=jnp.float32,
                   precision=jax.lax.Precision.HIGHEST)
    out_ref[...] = acc + base + bias_ref[...]


def kernel(self_input, self_adj, view2_input, view2_adj, view3_input,
           view3_adj, weight_self, weight_view2, weight_view3,
           weight_all_views, bias):
    c = (1.01 * weight_all_views.astype(jnp.float32)).reshape(3)
    w1s = weight_self * c[0]
    w2s = weight_view2 * c[1]
    w3s = weight_view3 * c[2]
    bias2d = bias.reshape(1, _F).astype(jnp.float32)

    full = pl.BlockSpec((_N, _F), lambda i: (0, 0))
    wspec = pl.BlockSpec((_F, _F), lambda i: (0, 0))
    adj_spec = pl.BlockSpec((_BM, _N), lambda i: (i, 0))
    row_spec = pl.BlockSpec((_BM, _F), lambda i: (i, 0))

    out = pl.pallas_call(
        _fused_body,
        grid=(_N // _BM,),
        in_specs=[full, full, full, wspec, wspec, wspec, wspec,
                  pl.BlockSpec((1, _F), lambda i: (0, 0)),
                  adj_spec, adj_spec, adj_spec],
        out_specs=row_spec,
        out_shape=jax.ShapeDtypeStruct((_N, _F), jnp.float32),
        scratch_shapes=[pltpu.VMEM((_N, _F), jnp.float32)] * 3,
        compiler_params=pltpu.CompilerParams(
            dimension_semantics=("arbitrary",),
        ),
    )(self_input, view2_input, view3_input, weight_self, w1s, w2s, w3s,
      bias2d, self_adj, view2_adj, view3_adj)

    return out
